# baseline (device time: 72826 ns/iter reference)
import jax
import jax.numpy as jnp
from jax import lax
from jax.experimental import pallas as pl
from jax.experimental.pallas import tpu as pltpu

N_DEV = 4
SQ = 256
SKV = 4096
HL = 8
DH = 128
DM = 1024
QB = 64
NC = 4
KPC = SKV // NC
NSB = KPC // QB
SCALE = 0.08838834764831843


def kernel(x, Wq, K_ext, V_ext, Wo):
    def body(x_ref, wq_ref, k_hbm, v_hbm, wo_ref, out_ref,
             kbuf, vbuf, comm, kv_sems, send_sems, recv_sems):
        p = lax.axis_index("i")
        right = lax.rem(p + 1, N_DEV)
        left = lax.rem(p + N_DEV - 1, N_DEV)

        def kv_copies():
            copies = []
            for c in range(NC):
                for sb in range(NSB):
                    row0 = QB * (NC * sb + c)
                    copies.append(pltpu.make_async_copy(
                        k_hbm.at[0, pl.ds(row0, QB), pl.ds(HL * p, HL), :],
                        kbuf.at[c, pl.ds(QB * sb, QB), :, :],
                        kv_sems.at[0, c]))
                    copies.append(pltpu.make_async_copy(
                        v_hbm.at[0, pl.ds(row0, QB), pl.ds(HL * p, HL), :],
                        vbuf.at[c, pl.ds(QB * sb, QB), :, :],
                        kv_sems.at[1, c]))
            return copies

        for cp in kv_copies():
            cp.start()

        barrier_sem = pltpu.get_barrier_semaphore()
        for nbr in [left, right]:
            pl.semaphore_signal(
                barrier_sem, inc=1,
                device_id=(nbr,), device_id_type=pl.DeviceIdType.MESH)
        pl.semaphore_wait(barrier_sem, 2)

        q = jnp.dot(x_ref[0].astype(jnp.bfloat16),
                    wq_ref[:, :].astype(jnp.bfloat16),
                    preferred_element_type=jnp.float32)
        q16 = q.astype(jnp.bfloat16)
        wo16 = wo_ref[:, :].astype(jnp.bfloat16)

        rdmas = {}

        def service(c, h):
            if h > 0:
                rdmas[(c, h - 1)].wait_recv()
            r = pltpu.make_async_remote_copy(
                src_ref=comm.at[c, h],
                dst_ref=comm.at[c, h + 1],
                send_sem=send_sems.at[c, h],
                recv_sem=recv_sems.at[c, h],
                device_id=(right,),
                device_id_type=pl.DeviceIdType.MESH)
            rdmas[(c, h)] = r
            r.start()

        waiters = kv_copies()
        for c in range(NC):
            for cp in waiters[2 * NSB * c:2 * NSB * (c + 1)]:
                cp.wait()
            qc = q16[QB * c:QB * (c + 1), :]
            ctx_parts = []
            for h in range(HL):
                qh = qc[:, h * DH:(h + 1) * DH]
                kh = kbuf[c, :, h, :].astype(jnp.bfloat16)
                vh = vbuf[c, :, h, :].astype(jnp.bfloat16)
                s = lax.dot_general(
                    qh, kh, (((1,), (1,)), ((), ())),
                    preferred_element_type=jnp.float32) * SCALE
                w = jnp.exp(s)
                d = jnp.sum(w, axis=1, keepdims=True)
                ctx_parts.append(
                    lax.dot_general(
                        w.astype(jnp.bfloat16), vh,
                        (((1,), (0,)), ((), ())),
                        preferred_element_type=jnp.float32) / d)
            ctx_c = jnp.concatenate(ctx_parts, axis=1)
            comm[c, 0] = jnp.dot(
                ctx_c.astype(jnp.bfloat16), wo16,
                preferred_element_type=jnp.float32)
            service(c, 0)
            if c >= 1:
                service(c - 1, 1)
            if c >= 2:
                service(c - 2, 2)

        service(2, 2)
        service(3, 1)
        service(3, 2)
        for c in range(NC):
            rdmas[(c, N_DEV - 2)].wait_recv()
            out_ref[0, QB * c:QB * (c + 1), :] = (
                (comm[c, 0] + comm[c, 1]) + (comm[c, 2] + comm[c, 3]))
        for r in rdmas.values():
            r.wait_send()

    return pl.pallas_call(
        body,
        out_shape=jax.ShapeDtypeStruct((1, SQ, DM), jnp.float32),
        in_specs=[
            pl.BlockSpec(memory_space=pltpu.VMEM),
            pl.BlockSpec(memory_space=pltpu.VMEM),
            pl.BlockSpec(memory_space=pl.ANY),
            pl.BlockSpec(memory_space=pl.ANY),
            pl.BlockSpec(memory_space=pltpu.VMEM),
        ],
        out_specs=pl.BlockSpec(memory_space=pltpu.VMEM),
        scratch_shapes=[
            pltpu.VMEM((NC, KPC, HL, DH), jnp.float32),
            pltpu.VMEM((NC, KPC, HL, DH), jnp.float32),
            pltpu.VMEM((NC, N_DEV, QB, DM), jnp.float32),
            pltpu.SemaphoreType.DMA((2, NC)),
            pltpu.SemaphoreType.DMA((NC, N_DEV - 1)),
            pltpu.SemaphoreType.DMA((NC, N_DEV - 1)),
        ],
        compiler_params=pltpu.CompilerParams(
            collective_id=0,
            vmem_limit_bytes=60 * 1024 * 1024,
        ),
    )(x, Wq, K_ext, V_ext, Wo)


# device time: 59431 ns/iter; 1.2254x vs baseline; 1.2254x over previous
import jax
import jax.numpy as jnp
from jax import lax
from jax.experimental import pallas as pl
from jax.experimental.pallas import tpu as pltpu

N_DEV = 4
SQ = 256
SKV = 4096
HL = 8
DH = 128
DM = 1024
QB = 64
NC = 4
KPC = SKV // NC
NSB = KPC // QB
SCALE = 0.08838834764831843


def kernel(x, Wq, K_ext, V_ext, Wo):
    def body(x_ref, wq_ref, k_hbm, v_hbm, wo_ref, out_ref,
             kbuf, vbuf, comm, kv_sems, send_sems, recv_sems):
        p = lax.axis_index("i")
        right = lax.rem(p + 1, N_DEV)
        left = lax.rem(p + N_DEV - 1, N_DEV)

        def kv_copies():
            copies = []
            for c in range(NC):
                for sb in range(NSB):
                    row0 = QB * (NC * sb + c)
                    copies.append(pltpu.make_async_copy(
                        k_hbm.at[0, pl.ds(row0, QB), pl.ds(HL * p, HL), :],
                        kbuf.at[c, pl.ds(QB * sb, QB), :, :],
                        kv_sems.at[0, c]))
                    copies.append(pltpu.make_async_copy(
                        v_hbm.at[0, pl.ds(row0, QB), pl.ds(HL * p, HL), :],
                        vbuf.at[c, pl.ds(QB * sb, QB), :, :],
                        kv_sems.at[1, c]))
            return copies

        for cp in kv_copies():
            cp.start()

        barrier_sem = pltpu.get_barrier_semaphore()
        for nbr in [left, right]:
            pl.semaphore_signal(
                barrier_sem, inc=1,
                device_id=(nbr,), device_id_type=pl.DeviceIdType.MESH)
        pl.semaphore_wait(barrier_sem, 2)

        q = jnp.dot(x_ref[0], wq_ref[:, :], preferred_element_type=jnp.float32)

        rdmas = {}

        def service(c, h):
            if h > 0:
                rdmas[(c, h - 1)].wait_recv()
            r = pltpu.make_async_remote_copy(
                src_ref=comm.at[c, h],
                dst_ref=comm.at[c, h + 1],
                send_sem=send_sems.at[c, h],
                recv_sem=recv_sems.at[c, h],
                device_id=(right,),
                device_id_type=pl.DeviceIdType.MESH)
            rdmas[(c, h)] = r
            r.start()

        waiters = kv_copies()
        for c in range(NC):
            for cp in waiters[2 * NSB * c:2 * NSB * (c + 1)]:
                cp.wait()
            qc = q[QB * c:QB * (c + 1), :]
            ctx_parts = []
            for h in range(HL):
                qh = qc[:, h * DH:(h + 1) * DH]
                kh = kbuf[c, :, h, :]
                vh = vbuf[c, :, h, :]
                s = lax.dot_general(
                    qh, kh, (((1,), (1,)), ((), ())),
                    preferred_element_type=jnp.float32) * SCALE
                w = jnp.exp(s)
                d = jnp.sum(w, axis=1, keepdims=True)
                ctx_parts.append(
                    jnp.dot(w, vh, preferred_element_type=jnp.float32) / d)
            ctx_c = jnp.concatenate(ctx_parts, axis=1)
            comm[c, 0] = jnp.dot(
                ctx_c, wo_ref[:, :], preferred_element_type=jnp.float32)
            service(c, 0)
            if c >= 1:
                service(c - 1, 1)
            if c >= 2:
                service(c - 2, 2)

        service(2, 2)
        service(3, 1)
        service(3, 2)
        for c in range(NC):
            rdmas[(c, N_DEV - 2)].wait_recv()
            out_ref[0, QB * c:QB * (c + 1), :] = (
                (comm[c, 0] + comm[c, 1]) + (comm[c, 2] + comm[c, 3]))
        for r in rdmas.values():
            r.wait_send()

    return pl.pallas_call(
        body,
        out_shape=jax.ShapeDtypeStruct((1, SQ, DM), jnp.float32),
        in_specs=[
            pl.BlockSpec(memory_space=pltpu.VMEM),
            pl.BlockSpec(memory_space=pltpu.VMEM),
            pl.BlockSpec(memory_space=pl.ANY),
            pl.BlockSpec(memory_space=pl.ANY),
            pl.BlockSpec(memory_space=pltpu.VMEM),
        ],
        out_specs=pl.BlockSpec(memory_space=pltpu.VMEM),
        scratch_shapes=[
            pltpu.VMEM((NC, KPC, HL, DH), jnp.float32),
            pltpu.VMEM((NC, KPC, HL, DH), jnp.float32),
            pltpu.VMEM((NC, N_DEV, QB, DM), jnp.float32),
            pltpu.SemaphoreType.DMA((2, NC)),
            pltpu.SemaphoreType.DMA((NC, N_DEV - 1)),
            pltpu.SemaphoreType.DMA((NC, N_DEV - 1)),
        ],
        compiler_params=pltpu.CompilerParams(
            collective_id=0,
            vmem_limit_bytes=60 * 1024 * 1024,
        ),
    )(x, Wq, K_ext, V_ext, Wo)


# device time: 48155 ns/iter; 1.5123x vs baseline; 1.2342x over previous
import jax
import jax.numpy as jnp
from jax import lax
from jax.experimental import pallas as pl
from jax.experimental.pallas import tpu as pltpu

N_DEV = 4
SQ = 256
SKV = 4096
HL = 8
DH = 128
DM = 1024
QB = 64
NC = 4
KPC = SKV // NC
NSB = KPC // QB
SCALE = 0.08838834764831843


def kernel(x, Wq, K_ext, V_ext, Wo):
    def body(x_ref, wq_ref, k_hbm, v_hbm, wo_ref, out_ref,
             kbuf, vbuf, comm, kv_sems, send_sems, recv_sems):
        p = lax.axis_index("i")
        right = lax.rem(p + 1, N_DEV)
        left = lax.rem(p + N_DEV - 1, N_DEV)

        def kv_copies():
            copies = []
            for c in range(NC):
                for sb in range(NSB):
                    row0 = QB * (NC * sb + c)
                    copies.append(pltpu.make_async_copy(
                        k_hbm.at[0, pl.ds(row0, QB), pl.ds(HL * p, HL), :],
                        kbuf.at[c, pl.ds(QB * sb, QB), :, :],
                        kv_sems.at[0, c]))
                    copies.append(pltpu.make_async_copy(
                        v_hbm.at[0, pl.ds(row0, QB), pl.ds(HL * p, HL), :],
                        vbuf.at[c, pl.ds(QB * sb, QB), :, :],
                        kv_sems.at[1, c]))
            return copies

        for cp in kv_copies():
            cp.start()

        barrier_sem = pltpu.get_barrier_semaphore()
        for nbr in [left, right]:
            pl.semaphore_signal(
                barrier_sem, inc=1,
                device_id=(nbr,), device_id_type=pl.DeviceIdType.MESH)
        pl.semaphore_wait(barrier_sem, 2)

        q = jnp.dot(x_ref[0], wq_ref[:, :], preferred_element_type=jnp.float32)

        rdmas = {}

        def service(c, h):
            if h > 0:
                rdmas[(c, h - 1)].wait_recv()
            r = pltpu.make_async_remote_copy(
                src_ref=comm.at[c, h],
                dst_ref=comm.at[c, h + 1],
                send_sem=send_sems.at[c, h],
                recv_sem=recv_sems.at[c, h],
                device_id=(right,),
                device_id_type=pl.DeviceIdType.MESH)
            rdmas[(c, h)] = r
            r.start()

        waiters = kv_copies()
        for c in range(NC):
            for cp in waiters[2 * NSB * c:2 * NSB * (c + 1)]:
                cp.wait()
            qc = q[QB * c:QB * (c + 1), :]
            ctx_parts = []
            for h in range(HL):
                qh = qc[:, h * DH:(h + 1) * DH]
                kh = kbuf[c, :, h, :]
                vh = vbuf[c, :, h, :]
                s = lax.dot_general(
                    qh, kh, (((1,), (1,)), ((), ())),
                    preferred_element_type=jnp.float32) * SCALE
                w = jnp.exp(s)
                d = jnp.sum(w, axis=1, keepdims=True)
                ctx_parts.append(
                    jnp.dot(w, vh, preferred_element_type=jnp.float32) / d)
            ctx_c = jnp.concatenate(ctx_parts, axis=1)
            comm[c, 0] = jnp.dot(
                ctx_c, wo_ref[:, :],
                preferred_element_type=jnp.float32).astype(jnp.bfloat16)
            service(c, 0)
            if c >= 1:
                service(c - 1, 1)
            if c >= 2:
                service(c - 2, 2)

        service(2, 2)
        service(3, 1)
        service(3, 2)
        for c in range(NC):
            rdmas[(c, N_DEV - 2)].wait_recv()
            out_ref[0, QB * c:QB * (c + 1), :] = (
                (comm[c, 0].astype(jnp.float32) +
                 comm[c, 1].astype(jnp.float32)) +
                (comm[c, 2].astype(jnp.float32) +
                 comm[c, 3].astype(jnp.float32)))
        for r in rdmas.values():
            r.wait_send()

    return pl.pallas_call(
        body,
        out_shape=jax.ShapeDtypeStruct((1, SQ, DM), jnp.float32),
        in_specs=[
            pl.BlockSpec(memory_space=pltpu.VMEM),
            pl.BlockSpec(memory_space=pltpu.VMEM),
            pl.BlockSpec(memory_space=pl.ANY),
            pl.BlockSpec(memory_space=pl.ANY),
            pl.BlockSpec(memory_space=pltpu.VMEM),
        ],
        out_specs=pl.BlockSpec(memory_space=pltpu.VMEM),
        scratch_shapes=[
            pltpu.VMEM((NC, KPC, HL, DH), jnp.float32),
            pltpu.VMEM((NC, KPC, HL, DH), jnp.float32),
            pltpu.VMEM((NC, N_DEV, QB, DM), jnp.bfloat16),
            pltpu.SemaphoreType.DMA((2, NC)),
            pltpu.SemaphoreType.DMA((NC, N_DEV - 1)),
            pltpu.SemaphoreType.DMA((NC, N_DEV - 1)),
        ],
        compiler_params=pltpu.CompilerParams(
            collective_id=0,
            vmem_limit_bytes=60 * 1024 * 1024,
        ),
    )(x, Wq, K_ext, V_ext, Wo)


# device time: 45462 ns/iter; 1.6019x vs baseline; 1.0592x over previous
import jax
import jax.numpy as jnp
from jax import lax
from jax.experimental import pallas as pl
from jax.experimental.pallas import tpu as pltpu

N_DEV = 4
SQ = 256
SKV = 4096
HL = 8
DH = 128
DM = 1024
QB = 64
NC = 4
KPC = SKV // NC
NSB = KPC // QB
SCALE = 0.08838834764831843


def kernel(x, Wq, K_ext, V_ext, Wo):
    def body(x_ref, wq_ref, k_hbm, v_hbm, wo_ref, out_ref,
             kbuf, vbuf, comm, kv_sems, send_sems, recv_sems):
        p = lax.axis_index("i")
        right = lax.rem(p + 1, N_DEV)
        left = lax.rem(p + N_DEV - 1, N_DEV)

        def kv_copies():
            copies = []
            for c in range(NC):
                for sb in range(NSB):
                    row0 = QB * (NC * sb + c)
                    copies.append(pltpu.make_async_copy(
                        k_hbm.at[0, pl.ds(row0, QB), pl.ds(HL * p, HL), :],
                        kbuf.at[c, pl.ds(QB * sb, QB), :, :],
                        kv_sems.at[0, c]))
                    copies.append(pltpu.make_async_copy(
                        v_hbm.at[0, pl.ds(row0, QB), pl.ds(HL * p, HL), :],
                        vbuf.at[c, pl.ds(QB * sb, QB), :, :],
                        kv_sems.at[1, c]))
            return copies

        for cp in kv_copies():
            cp.start()

        barrier_sem = pltpu.get_barrier_semaphore()
        for nbr in [left, right]:
            pl.semaphore_signal(
                barrier_sem, inc=1,
                device_id=(nbr,), device_id_type=pl.DeviceIdType.MESH)
        pl.semaphore_wait(barrier_sem, 2)

        q = jnp.dot(x_ref[0], wq_ref[:, :], preferred_element_type=jnp.float32)

        rdmas = {}

        def service(c, h):
            if h > 0:
                rdmas[(c, h - 1)].wait_recv()
            r = pltpu.make_async_remote_copy(
                src_ref=comm.at[c, h],
                dst_ref=comm.at[c, h + 1],
                send_sem=send_sems.at[c, h],
                recv_sem=recv_sems.at[c, h],
                device_id=(right,) if c % 2 == 0 else (left,),
                device_id_type=pl.DeviceIdType.MESH)
            rdmas[(c, h)] = r
            r.start()

        waiters = kv_copies()
        for c in range(NC):
            for cp in waiters[2 * NSB * c:2 * NSB * (c + 1)]:
                cp.wait()
            qc = q[QB * c:QB * (c + 1), :]
            ctx_parts = []
            for h in range(HL):
                qh = qc[:, h * DH:(h + 1) * DH]
                kh = kbuf[c, :, h, :]
                vh = vbuf[c, :, h, :]
                s = lax.dot_general(
                    qh, kh, (((1,), (1,)), ((), ())),
                    preferred_element_type=jnp.float32) * SCALE
                w = jnp.exp(s)
                d = jnp.sum(w, axis=1, keepdims=True)
                ctx_parts.append(
                    jnp.dot(w, vh, preferred_element_type=jnp.float32) / d)
            ctx_c = jnp.concatenate(ctx_parts, axis=1)
            comm[c, 0] = jnp.dot(
                ctx_c, wo_ref[:, :],
                preferred_element_type=jnp.float32).astype(jnp.bfloat16)
            service(c, 0)
            if c >= 1:
                service(c - 1, 1)
            if c >= 2:
                service(c - 2, 2)

        service(2, 2)
        service(3, 1)
        service(3, 2)
        for c in range(NC):
            rdmas[(c, N_DEV - 2)].wait_recv()
            out_ref[0, QB * c:QB * (c + 1), :] = (
                (comm[c, 0].astype(jnp.float32) +
                 comm[c, 1].astype(jnp.float32)) +
                (comm[c, 2].astype(jnp.float32) +
                 comm[c, 3].astype(jnp.float32)))
        for r in rdmas.values():
            r.wait_send()

    return pl.pallas_call(
        body,
        out_shape=jax.ShapeDtypeStruct((1, SQ, DM), jnp.float32),
        in_specs=[
            pl.BlockSpec(memory_space=pltpu.VMEM),
            pl.BlockSpec(memory_space=pltpu.VMEM),
            pl.BlockSpec(memory_space=pl.ANY),
            pl.BlockSpec(memory_space=pl.ANY),
            pl.BlockSpec(memory_space=pltpu.VMEM),
        ],
        out_specs=pl.BlockSpec(memory_space=pltpu.VMEM),
        scratch_shapes=[
            pltpu.VMEM((NC, KPC, HL, DH), jnp.float32),
            pltpu.VMEM((NC, KPC, HL, DH), jnp.float32),
            pltpu.VMEM((NC, N_DEV, QB, DM), jnp.bfloat16),
            pltpu.SemaphoreType.DMA((2, NC)),
            pltpu.SemaphoreType.DMA((NC, N_DEV - 1)),
            pltpu.SemaphoreType.DMA((NC, N_DEV - 1)),
        ],
        compiler_params=pltpu.CompilerParams(
            collective_id=0,
            vmem_limit_bytes=60 * 1024 * 1024,
        ),
    )(x, Wq, K_ext, V_ext, Wo)


# device time: 40533 ns/iter; 1.7967x vs baseline; 1.1216x over previous
import jax
import jax.numpy as jnp
from jax import lax
from jax.experimental import pallas as pl
from jax.experimental.pallas import tpu as pltpu

N_DEV = 4
SQ = 256
SKV = 4096
HL = 8
DH = 128
DM = 1024
QB = 64
NC = 4
KPC = SKV // NC
NSB = KPC // QB
SCALE = 0.08838834764831843


def kernel(x, Wq, K_ext, V_ext, Wo):
    def body(x_ref, wq_ref, k_hbm, v_hbm, wo_ref, out_ref,
             kbuf, vbuf, comm, kv_sems, send_sems, recv_sems):
        p = lax.axis_index("i")
        peers = [lax.rem(p + d, N_DEV) for d in range(1, N_DEV)]

        def kv_copies():
            copies = []
            for c in range(NC):
                for sb in range(NSB):
                    row0 = QB * (NC * sb + c)
                    copies.append(pltpu.make_async_copy(
                        k_hbm.at[0, pl.ds(row0, QB), pl.ds(HL * p, HL), :],
                        kbuf.at[c, pl.ds(QB * sb, QB), :, :],
                        kv_sems.at[0, c]))
                    copies.append(pltpu.make_async_copy(
                        v_hbm.at[0, pl.ds(row0, QB), pl.ds(HL * p, HL), :],
                        vbuf.at[c, pl.ds(QB * sb, QB), :, :],
                        kv_sems.at[1, c]))
            return copies

        for cp in kv_copies():
            cp.start()

        barrier_sem = pltpu.get_barrier_semaphore()
        for nbr in peers:
            pl.semaphore_signal(
                barrier_sem, inc=1,
                device_id=(nbr,), device_id_type=pl.DeviceIdType.MESH)
        pl.semaphore_wait(barrier_sem, N_DEV - 1)

        q = jnp.dot(x_ref[0], wq_ref[:, :], preferred_element_type=jnp.float32)

        rdmas = {}

        def broadcast(c):
            for d in range(1, N_DEV):
                r = pltpu.make_async_remote_copy(
                    src_ref=comm.at[c, 0],
                    dst_ref=comm.at[c, d],
                    send_sem=send_sems.at[c, d - 1],
                    recv_sem=recv_sems.at[c, d - 1],
                    device_id=(peers[d - 1],),
                    device_id_type=pl.DeviceIdType.MESH)
                rdmas[(c, d)] = r
                r.start()

        waiters = kv_copies()
        for c in range(NC):
            for cp in waiters[2 * NSB * c:2 * NSB * (c + 1)]:
                cp.wait()
            qc = q[QB * c:QB * (c + 1), :]
            ctx_parts = []
            for h in range(HL):
                qh = qc[:, h * DH:(h + 1) * DH]
                kh = kbuf[c, :, h, :]
                vh = vbuf[c, :, h, :]
                s = lax.dot_general(
                    qh, kh, (((1,), (1,)), ((), ())),
                    preferred_element_type=jnp.float32) * SCALE
                w = jnp.exp(s)
                d = jnp.sum(w, axis=1, keepdims=True)
                ctx_parts.append(
                    jnp.dot(w, vh, preferred_element_type=jnp.float32) / d)
            ctx_c = jnp.concatenate(ctx_parts, axis=1)
            comm[c, 0] = jnp.dot(
                ctx_c, wo_ref[:, :],
                preferred_element_type=jnp.float32).astype(jnp.bfloat16)
            broadcast(c)

        for c in range(NC):
            for d in range(1, N_DEV):
                rdmas[(c, d)].wait_recv()
            out_ref[0, QB * c:QB * (c + 1), :] = (
                (comm[c, 0].astype(jnp.float32) +
                 comm[c, 1].astype(jnp.float32)) +
                (comm[c, 2].astype(jnp.float32) +
                 comm[c, 3].astype(jnp.float32)))
        for r in rdmas.values():
            r.wait_send()

    return pl.pallas_call(
        body,
        out_shape=jax.ShapeDtypeStruct((1, SQ, DM), jnp.float32),
        in_specs=[
            pl.BlockSpec(memory_space=pltpu.VMEM),
            pl.BlockSpec(memory_space=pltpu.VMEM),
            pl.BlockSpec(memory_space=pl.ANY),
            pl.BlockSpec(memory_space=pl.ANY),
            pl.BlockSpec(memory_space=pltpu.VMEM),
        ],
        out_specs=pl.BlockSpec(memory_space=pltpu.VMEM),
        scratch_shapes=[
            pltpu.VMEM((NC, KPC, HL, DH), jnp.float32),
            pltpu.VMEM((NC, KPC, HL, DH), jnp.float32),
            pltpu.VMEM((NC, N_DEV, QB, DM), jnp.bfloat16),
            pltpu.SemaphoreType.DMA((2, NC)),
            pltpu.SemaphoreType.DMA((NC, N_DEV - 1)),
            pltpu.SemaphoreType.DMA((NC, N_DEV - 1)),
        ],
        compiler_params=pltpu.CompilerParams(
            collective_id=0,
            vmem_limit_bytes=60 * 1024 * 1024,
        ),
    )(x, Wq, K_ext, V_ext, Wo)
